# baseline (speedup 1.0000x reference)
import functools

import jax
import jax.numpy as jnp
from jax import lax
from jax.experimental import pallas as pl
from jax.experimental.pallas import tpu as pltpu
from jax.experimental.pallas import tpu_sc as plsc

N_NODES = 10000
D_FEAT = 256
N_EDGES = 160000

NC = 2
NS = 16
L = 16

HALF = N_NODES // NC
ACC_ROWS = 5120
ROWS_PER_TILE = ACC_ROWS // NS

E_PER_TILE = N_EDGES // NS
STAGE = 2000
N_STAGE = E_PER_TILE // STAGE
CHUNK = 32
CAP = STAGE + CHUNK
MAX_CH = CAP // CHUNK

OUT_CH = 8
OUT_ELEMS = OUT_CH * D_FEAT
N_OUT_CHUNKS = HALF // OUT_CH


def _body(x_hbm, tab_hbm, rows_a, cols_a, rows_b, cols_b, out_hbm,
          rbuf, cbuf, ccol, clrow, gbuf0, gbuf1, ibuf0, ibuf1, obuf, accf,
          gsem0, gsem1, ssem0, ssem1):
    gbufs, ibufs = (gbuf0, gbuf1), (ibuf0, ibuf1)
    gsems, ssems = (gsem0, gsem1), (ssem0, ssem1)
    c = lax.axis_index("c")
    t = lax.axis_index("s")
    base_row = c * HALF
    trash = HALF + t

    zero = jnp.zeros((L,), jnp.float32)

    for j in range(OUT_ELEMS // L):
        obuf[pl.ds(j * L, L)] = zero

    def _zero_chunk(k, _):
        off = (t * ROWS_PER_TILE + k * OUT_CH) * D_FEAT
        pltpu.sync_copy(obuf, accf.at[pl.ds(off, OUT_ELEMS)])
        return 0

    lax.fori_loop(0, ROWS_PER_TILE // OUT_CH, _zero_chunk, 0)
    plsc.subcore_barrier()

    def _edges(rows_hbm, cols_hbm):
        zero_i = jnp.zeros((L,), jnp.int32)
        trash_v = zero_i + trash

        def _round(s, _):
            off = t * E_PER_TILE + s * STAGE
            pltpu.sync_copy(rows_hbm.at[pl.ds(off, STAGE)], rbuf)
            pltpu.sync_copy(cols_hbm.at[pl.ds(off, STAGE)], cbuf)

            def _group(i, n):
                r = rbuf[pl.ds(i * L, L)]
                cv = cbuf[pl.ds(i * L, L)]
                local = r - base_row
                mask = (local >= 0) & (local < HALF)
                mv = jnp.where(mask, 1, 0).astype(jnp.int32)
                scan = plsc.cumsum(mv)
                pos = n + scan - 1
                plsc.store_scatter(ccol, [pos], cv, mask=mask)
                plsc.store_scatter(clrow, [pos], local, mask=mask)
                return n + jnp.squeeze(lax.slice(scan, (L - 1,), (L,)))

            n = lax.fori_loop(0, STAGE // L, _group, jnp.int32(0))

            def _pad(j, _):
                ccol[pl.ds(n + j * L, L)] = zero_i
                clrow[pl.ds(n + j * L, L)] = trash_v
                return 0

            lax.fori_loop(0, CHUNK // L, _pad, 0)
            n_chunks = (n + CHUNK - 1) // CHUNK

            def _fire_g(k, b):
                coff = k * CHUNK
                pltpu.async_copy(
                    x_hbm.at[ccol.at[pl.ds(coff, CHUNK)]], gbufs[b], gsems[b])
                pltpu.async_copy(
                    tab_hbm.at[clrow.at[pl.ds(coff, CHUNK)]], ibufs[b],
                    gsems[b])

            def _fire_s(b):
                pltpu.make_async_copy(
                    x_hbm.at[pl.ds(0, CHUNK)], gbufs[b], gsems[b]).wait()
                pltpu.make_async_copy(
                    tab_hbm.at[pl.ds(0, CHUNK)], ibufs[b], gsems[b]).wait()
                for e in range(CHUNK):
                    for h in range(2):
                        pltpu.async_copy(
                            gbufs[b].at[e, pl.ds(h * 128, 128)],
                            accf.at[ibufs[b].at[e, pl.ds(h * 128, 128)]],
                            ssems[b], add=True)

            def _drain_s(b):
                for e in range(CHUNK):
                    for h in range(2):
                        pltpu.make_async_copy(
                            gbufs[b].at[e, pl.ds(h * 128, 128)],
                            accf.at[ibufs[b].at[e, pl.ds(h * 128, 128)]],
                            ssems[b]).wait()

            def _pair(m, _):
                @pl.when(2 * m < n_chunks)
                def _():
                    _fire_g(2 * m, 0)

                @pl.when(2 * m + 1 < n_chunks)
                def _():
                    _fire_g(2 * m + 1, 1)

                @pl.when(2 * m < n_chunks)
                def _():
                    _fire_s(0)

                @pl.when(2 * m + 1 < n_chunks)
                def _():
                    _fire_s(1)

                @pl.when(2 * m < n_chunks)
                def _():
                    _drain_s(0)

                @pl.when(2 * m + 1 < n_chunks)
                def _():
                    _drain_s(1)

                return 0

            lax.fori_loop(0, (MAX_CH + 1) // 2, _pair, 0)
            return 0

        lax.fori_loop(0, N_STAGE, _round, 0)

    _edges(rows_a, cols_a)
    _edges(rows_b, cols_b)
    plsc.subcore_barrier()

    def _readout(k, _):
        g = k * NS + t

        @pl.when(g < N_OUT_CHUNKS)
        def _():
            pltpu.sync_copy(accf.at[pl.ds(g * OUT_ELEMS, OUT_ELEMS)], obuf)
            for j in range(OUT_ELEMS // L):
                v = obuf[pl.ds(j * L, L)]
                obuf[pl.ds(j * L, L)] = v * 0.5
            out_off = (base_row * D_FEAT) + g * OUT_ELEMS
            pltpu.sync_copy(obuf, out_hbm.at[pl.ds(out_off, OUT_ELEMS)])

        return 0

    lax.fori_loop(0, (N_OUT_CHUNKS + NS - 1) // NS, _readout, 0)


@jax.jit
def kernel(x, edge_index_a, edge_index_b):
    tab = jnp.arange(ACC_ROWS * D_FEAT, dtype=jnp.int32).reshape(
        ACC_ROWS, D_FEAT)
    call = pl.kernel(
        _body,
        out_type=jax.ShapeDtypeStruct((N_NODES * D_FEAT,), jnp.float32),
        mesh=plsc.VectorSubcoreMesh(core_axis_name="c", subcore_axis_name="s"),
        compiler_params=pltpu.CompilerParams(
            use_tc_tiling_on_sc=False, needs_layout_passes=False),
        scratch_types=[
            pltpu.VMEM((STAGE,), jnp.int32),
            pltpu.VMEM((STAGE,), jnp.int32),
            pltpu.VMEM((CAP,), jnp.int32),
            pltpu.VMEM((CAP,), jnp.int32),
            pltpu.VMEM((CHUNK, D_FEAT), jnp.float32),
            pltpu.VMEM((CHUNK, D_FEAT), jnp.float32),
            pltpu.VMEM((CHUNK, D_FEAT), jnp.int32),
            pltpu.VMEM((CHUNK, D_FEAT), jnp.int32),
            pltpu.VMEM((OUT_ELEMS,), jnp.float32),
            pltpu.VMEM_SHARED((ACC_ROWS * D_FEAT,), jnp.float32),
            pltpu.SemaphoreType.DMA,
            pltpu.SemaphoreType.DMA,
            pltpu.SemaphoreType.DMA,
            pltpu.SemaphoreType.DMA,
        ],
    )
    flat = call(x, tab, edge_index_a[0], edge_index_a[1],
                edge_index_b[0], edge_index_b[1])
    return flat.reshape(N_NODES, D_FEAT)


# speedup vs baseline: 1.2907x; 1.2907x over previous
import functools

import jax
import jax.numpy as jnp
from jax import lax
from jax.experimental import pallas as pl
from jax.experimental.pallas import tpu as pltpu
from jax.experimental.pallas import tpu_sc as plsc

N_NODES = 10000
D_FEAT = 256
N_EDGES = 160000

NC = 2
NS = 16
L = 16

HALF = N_NODES // NC
ACC_ROWS = 5120
ROWS_PER_TILE = ACC_ROWS // NS

E_PER_TILE = N_EDGES // NS
STAGE = 2000
N_STAGE = E_PER_TILE // STAGE
CHUNK = 32
CAP = STAGE + CHUNK
MAX_CH = CAP // CHUNK

OUT_CH = 8
OUT_ELEMS = OUT_CH * D_FEAT
N_OUT_CHUNKS = HALF // OUT_CH


def _body(x_hbm, tab_hbm, rows_a, cols_a, rows_b, cols_b, out_hbm,
          rbuf, cbuf, ccol, clrow, gbuf0, gbuf1, ibuf0, ibuf1, obuf, accf,
          gsem0, gsem1, ssem0, ssem1):
    gbufs, ibufs = (gbuf0, gbuf1), (ibuf0, ibuf1)
    gsems, ssems = (gsem0, gsem1), (ssem0, ssem1)
    c = lax.axis_index("c")
    t = lax.axis_index("s")
    base_row = c * HALF
    trash = HALF + t

    zero = jnp.zeros((L,), jnp.float32)

    for j in range(OUT_ELEMS // L):
        obuf[pl.ds(j * L, L)] = zero

    def _zero_chunk(k, _):
        off = (t * ROWS_PER_TILE + k * OUT_CH) * D_FEAT
        pltpu.sync_copy(obuf, accf.at[pl.ds(off, OUT_ELEMS)])
        return 0

    lax.fori_loop(0, ROWS_PER_TILE // OUT_CH, _zero_chunk, 0)
    plsc.subcore_barrier()

    def _edges(rows_hbm, cols_hbm):
        zero_i = jnp.zeros((L,), jnp.int32)
        trash_v = zero_i + trash

        def _round(s, _):
            off = t * E_PER_TILE + s * STAGE
            pltpu.sync_copy(rows_hbm.at[pl.ds(off, STAGE)], rbuf)
            pltpu.sync_copy(cols_hbm.at[pl.ds(off, STAGE)], cbuf)

            def _group(i, n):
                r = rbuf[pl.ds(i * L, L)]
                cv = cbuf[pl.ds(i * L, L)]
                local = r - base_row
                mask = (local >= 0) & (local < HALF)
                mv = jnp.where(mask, 1, 0).astype(jnp.int32)
                scan = plsc.cumsum(mv)
                pos = n + scan - 1
                plsc.store_scatter(ccol, [pos], cv, mask=mask)
                plsc.store_scatter(clrow, [pos], local, mask=mask)
                return n + jnp.squeeze(lax.slice(scan, (L - 1,), (L,)))

            n = lax.fori_loop(0, STAGE // L, _group, jnp.int32(0))

            def _pad(j, _):
                ccol[pl.ds(n + j * L, L)] = zero_i
                clrow[pl.ds(n + j * L, L)] = trash_v
                return 0

            lax.fori_loop(0, CHUNK // L, _pad, 0)
            n_chunks = (n + CHUNK - 1) // CHUNK

            def _fire_g(k, b):
                coff = k * CHUNK
                pltpu.async_copy(
                    x_hbm.at[ccol.at[pl.ds(coff, CHUNK)]], gbufs[b], gsems[b])
                pltpu.async_copy(
                    tab_hbm.at[clrow.at[pl.ds(coff, CHUNK)]], ibufs[b],
                    gsems[b])

            def _fire_s(b):
                pltpu.make_async_copy(
                    x_hbm.at[pl.ds(0, CHUNK)], gbufs[b], gsems[b]).wait()
                pltpu.make_async_copy(
                    tab_hbm.at[pl.ds(0, CHUNK)], ibufs[b], gsems[b]).wait()
                for e in range(CHUNK):
                    pltpu.async_copy(
                        gbufs[b].at[e], accf.at[ibufs[b].at[e]], ssems[b],
                        add=True)

            def _drain_s(b):
                for e in range(CHUNK):
                    pltpu.make_async_copy(
                        gbufs[b].at[e], accf.at[ibufs[b].at[e]],
                        ssems[b]).wait()

            @pl.when(0 < n_chunks)
            def _():
                _fire_g(0, 0)

            def _pair(m, _):
                @pl.when((m > 0) & (2 * m <= n_chunks))
                def _():
                    _drain_s(1)

                @pl.when(2 * m + 1 < n_chunks)
                def _():
                    _fire_g(2 * m + 1, 1)

                @pl.when(2 * m < n_chunks)
                def _():
                    _fire_s(0)

                @pl.when(2 * m + 1 < n_chunks)
                def _():
                    _drain_s(0)

                @pl.when(2 * m + 2 < n_chunks)
                def _():
                    _fire_g(2 * m + 2, 0)

                @pl.when(2 * m + 1 < n_chunks)
                def _():
                    _fire_s(1)

                return 0

            lax.fori_loop(0, (MAX_CH + 1) // 2, _pair, 0)
            @pl.when(n_chunks % 2 == 1)
            def _():
                _drain_s(0)

            return 0

        lax.fori_loop(0, N_STAGE, _round, 0)

    _edges(rows_a, cols_a)
    _edges(rows_b, cols_b)
    plsc.subcore_barrier()

    def _readout(k, _):
        g = k * NS + t

        @pl.when(g < N_OUT_CHUNKS)
        def _():
            pltpu.sync_copy(accf.at[pl.ds(g * OUT_ELEMS, OUT_ELEMS)], obuf)
            for j in range(OUT_ELEMS // L):
                v = obuf[pl.ds(j * L, L)]
                obuf[pl.ds(j * L, L)] = v * 0.5
            out_off = (base_row * D_FEAT) + g * OUT_ELEMS
            pltpu.sync_copy(obuf, out_hbm.at[pl.ds(out_off, OUT_ELEMS)])

        return 0

    lax.fori_loop(0, (N_OUT_CHUNKS + NS - 1) // NS, _readout, 0)


@jax.jit
def kernel(x, edge_index_a, edge_index_b):
    tab = jnp.arange(ACC_ROWS * D_FEAT, dtype=jnp.int32).reshape(
        ACC_ROWS, D_FEAT)
    call = pl.kernel(
        _body,
        out_type=jax.ShapeDtypeStruct((N_NODES * D_FEAT,), jnp.float32),
        mesh=plsc.VectorSubcoreMesh(core_axis_name="c", subcore_axis_name="s"),
        compiler_params=pltpu.CompilerParams(
            use_tc_tiling_on_sc=False, needs_layout_passes=False),
        scratch_types=[
            pltpu.VMEM((STAGE,), jnp.int32),
            pltpu.VMEM((STAGE,), jnp.int32),
            pltpu.VMEM((CAP,), jnp.int32),
            pltpu.VMEM((CAP,), jnp.int32),
            pltpu.VMEM((CHUNK, D_FEAT), jnp.float32),
            pltpu.VMEM((CHUNK, D_FEAT), jnp.float32),
            pltpu.VMEM((CHUNK, D_FEAT), jnp.int32),
            pltpu.VMEM((CHUNK, D_FEAT), jnp.int32),
            pltpu.VMEM((OUT_ELEMS,), jnp.float32),
            pltpu.VMEM_SHARED((ACC_ROWS * D_FEAT,), jnp.float32),
            pltpu.SemaphoreType.DMA,
            pltpu.SemaphoreType.DMA,
            pltpu.SemaphoreType.DMA,
            pltpu.SemaphoreType.DMA,
        ],
    )
    flat = call(x, tab, edge_index_a[0], edge_index_a[1],
                edge_index_b[0], edge_index_b[1])
    return flat.reshape(N_NODES, D_FEAT)
